# baseline (speedup 1.0000x reference)
import functools

import jax
import jax.numpy as jnp
from jax import lax
from jax.experimental import pallas as pl
from jax.experimental.pallas import tpu as pltpu
from jax.experimental.pallas import tpu_sc as plsc


def _gather_call(x_flat, w_packed, num_feat, rows_per_w, chunk, d):
    n_chunks = rows_per_w // chunk
    total_rows = x_flat.shape[0]
    dh = w_packed.shape[1]
    vocab = w_packed.shape[0] // num_feat
    lanes = 16

    mesh = plsc.VectorSubcoreMesh(core_axis_name="c", subcore_axis_name="s")

    @functools.partial(
        pl.kernel,
        mesh=mesh,
        compiler_params=pltpu.CompilerParams(
            use_tc_tiling_on_sc=False, needs_layout_passes=False),
        out_type=jax.ShapeDtypeStruct((total_rows, d), jnp.float32),
        scratch_types=(
            [pltpu.VMEM((chunk,), jnp.int32) for _ in range(2)]
            + [pltpu.VMEM((chunk,), jnp.int32)]
            + [pltpu.VMEM((chunk, dh), jnp.int32) for _ in range(2)]
            + [pltpu.VMEM((chunk, d), jnp.float32) for _ in range(2)]
            + [pltpu.SemaphoreType.DMA for _ in range(6)]
        ),
    )
    def k(x_hbm, w_hbm, out_hbm, idx0, idx1, offs_v, pk0, pk1, f0, f1,
          semi0, semi1, semg0, semg1, semo0, semo1):
        idx_b = (idx0, idx1)
        pk_b = (pk0, pk1)
        f_b = (f0, f1)
        semi = (semi0, semi1)
        semg = (semg0, semg1)
        semo = (semo0, semo1)

        wid = lax.axis_index("s") * 2 + lax.axis_index("c")
        wbase = wid * rows_per_w

        def fill_offs(i, _):
            sl = pl.ds(i * lanes, lanes)
            v = lax.iota(jnp.int32, lanes) + i * lanes
            offs_v[sl] = lax.rem(v, num_feat) * vocab
            return 0

        lax.fori_loop(0, chunk // lanes, fill_offs, 0)

        def row_slice(c):
            return pl.ds(wbase + c * chunk, chunk)

        def enqueue_gathers(b):
            idx_v = idx_b[b]

            def body(g, _):
                sl = pl.ds(g * lanes, lanes)
                v = idx_v[sl] + offs_v[sl]
                pltpu.async_copy(w_hbm.at[v], pk_b[b].at[sl], semg[b])
                return 0

            lax.fori_loop(0, chunk // lanes, body, 0)

        def drain_gathers(b):
            pltpu.make_async_copy(
                w_hbm.at[pl.ds(0, chunk)], pk_b[b], semg[b]).wait()

        lane = lax.iota(jnp.int32, lanes)
        ev = lane * 2
        od = ev + 1

        def convert_chunk(b):
            pk_v, f_v = pk_b[b], f_b[b]

            hi_mask = jnp.full((lanes,), -65536, jnp.int32)

            def body(r, _):
                rsplat = jnp.full((lanes,), r, jnp.int32)
                packed = plsc.load_gather(pk_v, [rsplat, lane])
                a = plsc.bitcast(lax.shift_left(packed, 16), jnp.float32)
                bvals = plsc.bitcast(
                    lax.bitwise_and(packed, hi_mask), jnp.float32)
                plsc.store_scatter(f_v, [rsplat, ev], a)
                plsc.store_scatter(f_v, [rsplat, od], bvals)
                return 0

            lax.fori_loop(0, chunk, body, 0)

        idx_d = [None] * n_chunks
        out_d = [None] * n_chunks
        idx_d[0] = pltpu.async_copy(x_hbm.at[row_slice(0)], idx_b[0], semi[0])
        for c in range(n_chunks):
            b = c % 2
            idx_d[c].wait()
            if c + 1 < n_chunks:
                nb = (c + 1) % 2
                idx_d[c + 1] = pltpu.async_copy(
                    x_hbm.at[row_slice(c + 1)], idx_b[nb], semi[nb])
            if c >= 2:
                out_d[c - 2].wait()
            enqueue_gathers(b)
            if c >= 1:
                drain_gathers(1 - b)
                convert_chunk(1 - b)
                out_d[c - 1] = pltpu.async_copy(
                    f_b[1 - b], out_hbm.at[row_slice(c - 1)], semo[1 - b])
        last_b = (n_chunks - 1) % 2
        drain_gathers(last_b)
        convert_chunk(last_b)
        out_d[n_chunks - 1] = pltpu.async_copy(
            f_b[last_b], out_hbm.at[row_slice(n_chunks - 1)], semo[last_b])
        out_d[n_chunks - 2].wait()
        out_d[n_chunks - 1].wait()

    return k(x_flat, w_packed)


def kernel(x, W):
    num_feat, vocab, d = W.shape
    batch = x.shape[0]
    total_rows = batch * num_feat

    nw = 32
    rows_per_w = total_rows // nw
    chunk = 832

    x_flat = x.reshape(total_rows).astype(jnp.int32)
    w_packed = jax.lax.bitcast_convert_type(
        W.astype(jnp.bfloat16).reshape(num_feat * vocab, d // 2, 2),
        jnp.int32)
    out = _gather_call(x_flat, w_packed, num_feat, rows_per_w, chunk, d)
    return out.reshape(batch, num_feat, d)


# speedup vs baseline: 2.1637x; 2.1637x over previous
import functools

import jax
import jax.numpy as jnp
from jax import lax
from jax.experimental import pallas as pl
from jax.experimental.pallas import tpu as pltpu
from jax.experimental.pallas import tpu_sc as plsc


def _gather_call(x_flat, w_flat, num_feat, rows_per_w, chunk):
    n_chunks = rows_per_w // chunk
    total_rows = x_flat.shape[0]
    d = w_flat.shape[1]
    vocab = w_flat.shape[0] // num_feat
    lanes = 16

    mesh = plsc.VectorSubcoreMesh(core_axis_name="c", subcore_axis_name="s")

    @functools.partial(
        pl.kernel,
        mesh=mesh,
        compiler_params=pltpu.CompilerParams(use_tc_tiling_on_sc=False),
        out_type=jax.ShapeDtypeStruct((total_rows, d), jnp.float32),
        scratch_types=(
            [pltpu.VMEM((chunk,), jnp.int32) for _ in range(2)]
            + [pltpu.VMEM((chunk,), jnp.int32)]
            + [pltpu.VMEM((chunk, d), jnp.float32) for _ in range(2)]
            + [pltpu.SemaphoreType.DMA for _ in range(6)]
        ),
    )
    def k(x_hbm, w_hbm, out_hbm, idx0, idx1, offs_v, rows0, rows1,
          semi0, semi1, semg0, semg1, semo0, semo1):
        idx_b = (idx0, idx1)
        rows_b = (rows0, rows1)
        semi = (semi0, semi1)
        semg = (semg0, semg1)
        semo = (semo0, semo1)

        wid = lax.axis_index("s") * 2 + lax.axis_index("c")
        wbase = wid * rows_per_w

        def fill_offs(i, _):
            sl = pl.ds(i * lanes, lanes)
            v = lax.iota(jnp.int32, lanes) + i * lanes
            offs_v[sl] = lax.rem(v, num_feat) * vocab
            return 0

        lax.fori_loop(0, chunk // lanes, fill_offs, 0)

        def row_slice(c):
            return pl.ds(wbase + c * chunk, chunk)

        def enqueue_gathers(b):
            idx_v, rows_v = idx_b[b], rows_b[b]

            def body(g, _):
                sl = pl.ds(g * lanes, lanes)
                v = idx_v[sl] + offs_v[sl]
                pltpu.async_copy(w_hbm.at[v], rows_v.at[sl], semg[b])
                return 0

            lax.fori_loop(0, chunk // lanes, body, 0)

        def drain_gathers(b):
            pltpu.make_async_copy(
                w_hbm.at[pl.ds(0, chunk)], rows_b[b], semg[b]).wait()

        idx_d = [None] * n_chunks
        out_d = [None] * n_chunks
        idx_d[0] = pltpu.async_copy(x_hbm.at[row_slice(0)], idx_b[0], semi[0])
        for c in range(n_chunks):
            b = c % 2
            idx_d[c].wait()
            if c + 1 < n_chunks:
                nb = (c + 1) % 2
                idx_d[c + 1] = pltpu.async_copy(
                    x_hbm.at[row_slice(c + 1)], idx_b[nb], semi[nb])
            if c >= 2:
                out_d[c - 2].wait()
            enqueue_gathers(b)
            if c >= 1:
                drain_gathers(1 - b)
                out_d[c - 1] = pltpu.async_copy(
                    rows_b[1 - b], out_hbm.at[row_slice(c - 1)], semo[1 - b])
        last_b = (n_chunks - 1) % 2
        drain_gathers(last_b)
        out_d[n_chunks - 1] = pltpu.async_copy(
            rows_b[last_b], out_hbm.at[row_slice(n_chunks - 1)], semo[last_b])
        out_d[n_chunks - 2].wait()
        out_d[n_chunks - 1].wait()

    return k(x_flat, w_flat)


def kernel(x, W):
    num_feat, vocab, d = W.shape
    batch = x.shape[0]
    total_rows = batch * num_feat

    nw = 32
    rows_per_w = total_rows // nw
    chunk = 1664

    x_flat = x.reshape(total_rows).astype(jnp.int32)
    w_flat = W.reshape(num_feat * vocab, d)
    out = _gather_call(x_flat, w_flat, num_feat, rows_per_w, chunk)
    return out.reshape(batch, num_feat, d)
